# baseline (speedup 1.0000x reference)
import functools

import jax
import jax.numpy as jnp
from jax import lax
from jax.experimental import pallas as pl
from jax.experimental.pallas import tpu as pltpu
from jax.experimental.pallas import tpu_sc as plsc

N_FRAMES = 1000
BATCH = 16384
L = 16
NC, NS = 2, 16
NW = NC * NS
BPW = BATCH // NW
GROUPS = BPW // L


def _body(idx_hbm, rays_hbm, mask_hbm, table_hbm, out_hbm,
          idx_v, mask_v, rays_v, table_v, out_v, sem):
    wid = lax.axis_index("s") * NC + lax.axis_index("c")
    base = wid * BPW

    cps = [
        pltpu.make_async_copy(table_hbm, table_v, sem),
        pltpu.make_async_copy(idx_hbm.at[pl.ds(base, BPW)], idx_v, sem),
        pltpu.make_async_copy(mask_hbm.at[pl.ds(base, BPW)], mask_v, sem),
        pltpu.make_async_copy(rays_hbm.at[:, pl.ds(base, BPW)], rays_v, sem),
    ]
    for cp in cps:
        cp.start()
    for cp in cps:
        cp.wait()

    zero = jnp.zeros((L,), jnp.float32)
    one = jnp.ones((L,), jnp.float32)
    for g in range(GROUPS):
        sl = pl.ds(g * L, L)
        iv = idx_v[sl]
        keep = mask_v[sl] == 1

        def tcol(c, fallback):
            raw = plsc.load_gather(table_v, [jnp.full((L,), c, jnp.int32), iv])
            return jnp.where(keep, raw, fallback)

        tx, ty, tz = tcol(0, zero), tcol(1, zero), tcol(2, zero)
        qx, qy, qz = tcol(3, zero), tcol(4, zero), tcol(5, zero)
        qw = tcol(6, one)
        ox, oy, oz = rays_v[0, sl], rays_v[1, sl], rays_v[2, sl]
        dx, dy, dz = rays_v[3, sl], rays_v[4, sl], rays_v[5, sl]

        r00 = 1.0 - 2.0 * (qy * qy + qz * qz)
        r01 = 2.0 * (qx * qy - qz * qw)
        r02 = 2.0 * (qx * qz + qy * qw)
        r10 = 2.0 * (qx * qy + qz * qw)
        r11 = 1.0 - 2.0 * (qx * qx + qz * qz)
        r12 = 2.0 * (qy * qz - qx * qw)
        r20 = 2.0 * (qx * qz - qy * qw)
        r21 = 2.0 * (qy * qz + qx * qw)
        r22 = 1.0 - 2.0 * (qx * qx + qy * qy)

        out_v[0, sl] = ox + tx
        out_v[1, sl] = oy + ty
        out_v[2, sl] = oz + tz
        out_v[3, sl] = r00 * dx + r01 * dy + r02 * dz
        out_v[4, sl] = r10 * dx + r11 * dy + r12 * dz
        out_v[5, sl] = r20 * dx + r21 * dy + r22 * dz

    pltpu.sync_copy(out_v, out_hbm.at[:, pl.ds(base, BPW)])


@jax.jit
def _run(idx, rays_t, mask, table_t):
    mesh = plsc.VectorSubcoreMesh(core_axis_name="c", subcore_axis_name="s")
    fn = functools.partial(
        pl.kernel,
        mesh=mesh,
        out_type=jax.ShapeDtypeStruct((6, BATCH), jnp.float32),
        compiler_params=pltpu.CompilerParams(needs_layout_passes=False),
        scratch_types=[
            pltpu.VMEM((BPW,), jnp.int32),
            pltpu.VMEM((BPW,), jnp.int32),
            pltpu.VMEM((6, BPW), jnp.float32),
            pltpu.VMEM((7, N_FRAMES), jnp.float32),
            pltpu.VMEM((6, BPW), jnp.float32),
            pltpu.SemaphoreType.DMA,
        ],
    )(_body)
    return fn(idx, rays_t, mask, table_t)


def kernel(image_indices, rays, depth_mask, correction_dict):
    idx = image_indices.astype(jnp.int32)
    mask = depth_mask.reshape(BATCH).astype(jnp.int32)
    out_t = _run(idx, rays.T, mask, correction_dict.T)
    return out_t.T
